# baseline (speedup 1.0000x reference)
import functools

import jax
import jax.numpy as jnp
from jax import lax
from jax.experimental import pallas as pl
from jax.experimental.pallas import tpu as pltpu
from jax.experimental.pallas import tpu_sc as plsc

_EPS = 0.1
_K = 10
_LN = 128


def _pack_body(nm_ref, out_ref, *, n_words):
    acc = nm_ref[:, 0:_LN]
    for k in range(1, 32):
        acc = acc | (nm_ref[:, k * _LN:(k + 1) * _LN] << k)
    out_ref[...] = acc


def _pack(nearest_map):
    n_cls = nearest_map.shape[1]
    rv = 512
    return pl.pallas_call(
        functools.partial(_pack_body, n_words=_LN),
        grid=(nearest_map.shape[0] // rv,),
        in_specs=[pl.BlockSpec((rv, n_cls), lambda i: (i, 0))],
        out_specs=pl.BlockSpec((rv, _LN), lambda i: (i, 0)),
        out_shape=jax.ShapeDtypeStruct((nearest_map.shape[0], _LN), jnp.int32),
        compiler_params=pltpu.CompilerParams(
            dimension_semantics=("parallel",),
        ),
    )(nearest_map)


def _make_sc_gather(n_rows, n_cls):
    info = plsc.get_sparse_core_info()
    nw = info.num_cores * info.num_subcores
    b_per_w = n_rows // nw
    chunk = 128
    n_chunks = b_per_w // chunk
    mesh = plsc.VectorSubcoreMesh(core_axis_name="c", subcore_axis_name="s")

    @functools.partial(
        pl.kernel, mesh=mesh,
        out_type=(
            jax.ShapeDtypeStruct((n_rows, _LN), jnp.int32),
            jax.ShapeDtypeStruct((n_rows, _LN), jnp.float32),
        ),
        scratch_types=[
            pltpu.VMEM((chunk,), jnp.int32),
            pltpu.VMEM((chunk, _LN), jnp.int32),
            pltpu.VMEM((chunk,), jnp.int32),
            pltpu.VMEM((chunk, _LN), jnp.float32),
            pltpu.SemaphoreType.DMA,
        ],
    )
    def sc_gather(packed_hbm, t_hbm, xrows_hbm, outg_hbm, outx_hbm,
                  idx_v, rows_v, fidx_v, xt_v, sem):
        wid = lax.axis_index("s") * info.num_cores + lax.axis_index("c")
        base = wid * b_per_w
        n_sub = n_cls // _LN

        def body(ci, carry):
            off = base + ci * chunk
            pltpu.sync_copy(t_hbm.at[pl.ds(off, chunk)], idx_v)
            pltpu.async_copy(packed_hbm.at[idx_v], rows_v, sem).wait()
            pltpu.sync_copy(rows_v, outg_hbm.at[pl.ds(off, chunk)])
            lane = lax.iota(jnp.int32, 16)
            for j in range(chunk // 16):
                t16 = idx_v[pl.ds(16 * j, 16)]
                fidx_v[pl.ds(16 * j, 16)] = (
                    (off + 16 * j + lane) * n_sub + (t16 >> 7))
            pltpu.async_copy(xrows_hbm.at[fidx_v], xt_v, sem).wait()
            pltpu.sync_copy(xt_v, outx_hbm.at[pl.ds(off, chunk)])
            return carry

        lax.fori_loop(0, n_chunks, body, 0)

    return sc_gather


def _block_body(x_ref, t2_ref, gp_ref, xt_ref, out_ref, *, rows, n_cls):
    x = x_ref[...]
    gp = gp_ref[...]
    tv = t2_ref[0]
    xrow = xt_ref[...]

    m = jnp.max(x, axis=1, keepdims=True)
    lse = m + jnp.log(jnp.sum(jnp.exp(x - m), axis=1, keepdims=True))

    thi = tv >> 7
    tlo = tv & (_LN - 1)
    colw = jax.lax.broadcasted_iota(jnp.int32, (rows, _LN), 1)
    lmask = colw == tlo
    gsh = (gp >> thi) & 1
    gt = jnp.sum(jnp.where(lmask, gsh, 0), axis=1, keepdims=True)
    gt = gt.astype(jnp.float32)
    xt = jnp.sum(jnp.where(lmask, xrow, 0.0), axis=1, keepdims=True)

    dot_acc = jnp.zeros((rows, _LN), jnp.float32)
    for k in range(32):
        sel = (gp << (31 - k)) < 0
        dot_acc = dot_acc + jnp.where(sel, x[:, k * _LN:(k + 1) * _LN], 0.0)
    dot = jnp.sum(dot_acc, axis=1, keepdims=True)
    cnt = jnp.sum(jax.lax.population_count(gp), axis=1,
                  keepdims=True).astype(jnp.float32)

    a = 1.0 - _EPS + _EPS / _K
    b = 2.0 * _EPS / _K
    c = _EPS / _K
    rowloss = -(a - b * gt) * (xt - lse) - c * (dot - cnt * lse)
    block_sum = jnp.sum(rowloss)

    @pl.when(pl.program_id(0) == 0)
    def _():
        out_ref[...] = jnp.zeros_like(out_ref)

    out_ref[...] = out_ref[...] + block_sum


def kernel(inputs, targets, nearest_map):
    bsz, n_cls = inputs.shape
    rows = 256 if bsz % 256 == 0 else bsz
    nblk = bsz // rows

    t2 = targets.reshape(nblk, rows, 1)
    packed = _pack(nearest_map)
    xrows = inputs.reshape(bsz * (n_cls // _LN), _LN)
    gathered, xtv = _make_sc_gather(bsz, n_cls)(packed, targets, xrows)

    total = pl.pallas_call(
        functools.partial(_block_body, rows=rows, n_cls=n_cls),
        grid=(nblk,),
        in_specs=[
            pl.BlockSpec((rows, n_cls), lambda i: (i, 0)),
            pl.BlockSpec((1, rows, 1), lambda i: (i, 0, 0)),
            pl.BlockSpec((rows, _LN), lambda i: (i, 0)),
            pl.BlockSpec((rows, _LN), lambda i: (i, 0)),
        ],
        out_specs=pl.BlockSpec((1, 1), lambda i: (0, 0)),
        out_shape=jax.ShapeDtypeStruct((1, 1), jnp.float32),
        compiler_params=pltpu.CompilerParams(
            dimension_semantics=("arbitrary",),
            vmem_limit_bytes=100 * 1024 * 1024,
        ),
    )(inputs, t2, gathered, xtv)

    return total[0, 0] * (1.0 / bsz)


# speedup vs baseline: 1.9130x; 1.9130x over previous
import functools

import jax
import jax.numpy as jnp
from jax import lax
from jax.experimental import pallas as pl
from jax.experimental.pallas import tpu as pltpu
from jax.experimental.pallas import tpu_sc as plsc

_EPS = 0.1
_K = 10
_LN = 128


def _pack_body(nm_ref, out_ref, *, n_words):
    acc = nm_ref[:, 0:_LN]
    for k in range(1, 32):
        acc = acc | (nm_ref[:, k * _LN:(k + 1) * _LN] << k)
    out_ref[...] = acc


def _pack(nearest_map):
    n_cls = nearest_map.shape[1]
    rv = 512
    return pl.pallas_call(
        functools.partial(_pack_body, n_words=_LN),
        grid=(nearest_map.shape[0] // rv,),
        in_specs=[pl.BlockSpec((rv, n_cls), lambda i: (i, 0))],
        out_specs=pl.BlockSpec((rv, _LN), lambda i: (i, 0)),
        out_shape=jax.ShapeDtypeStruct((nearest_map.shape[0], _LN), jnp.int32),
        compiler_params=pltpu.CompilerParams(
            dimension_semantics=("parallel",),
        ),
    )(nearest_map)


def _make_sc_gather(n_rows):
    info = plsc.get_sparse_core_info()
    nw = info.num_cores * info.num_subcores
    b_per_w = n_rows // nw
    chunk = 128
    n_chunks = b_per_w // chunk
    mesh = plsc.VectorSubcoreMesh(core_axis_name="c", subcore_axis_name="s")

    @functools.partial(
        pl.kernel, mesh=mesh,
        out_type=jax.ShapeDtypeStruct((n_rows, _LN), jnp.int32),
        scratch_types=[
            pltpu.VMEM((chunk,), jnp.int32),
            pltpu.VMEM((chunk, _LN), jnp.int32),
            pltpu.SemaphoreType.DMA,
        ],
    )
    def sc_gather(packed_hbm, t_hbm, out_hbm, idx_v, rows_v, sem):
        wid = lax.axis_index("s") * info.num_cores + lax.axis_index("c")
        base = wid * b_per_w

        def body(ci, carry):
            off = base + ci * chunk
            pltpu.sync_copy(t_hbm.at[pl.ds(off, chunk)], idx_v)
            pltpu.async_copy(packed_hbm.at[idx_v], rows_v, sem).wait()
            pltpu.sync_copy(rows_v, out_hbm.at[pl.ds(off, chunk)])
            return carry

        lax.fori_loop(0, n_chunks, body, 0)

    return sc_gather


def _block_body(x_ref, t2_ref, gp_ref, out_ref, *, rows, n_cls):
    x = x_ref[...]
    gp = gp_ref[...]
    tv = t2_ref[0]

    log2e = 1.4426950408889634
    m = jnp.max(x, axis=1, keepdims=True)
    e = jnp.exp2(x * log2e - m * log2e)
    lse = m + jnp.log(jnp.sum(e, axis=1, keepdims=True))

    thi = tv >> 7
    tlo = tv & (_LN - 1)

    dot_acc = jnp.zeros((rows, _LN), jnp.float32)
    xrow_acc = jnp.zeros((rows, _LN), jnp.float32)
    for k in range(32):
        xk = x[:, k * _LN:(k + 1) * _LN]
        sel = (gp << (31 - k)) < 0
        dot_acc = dot_acc + jnp.where(sel, xk, 0.0)
        xrow_acc = xrow_acc + jnp.where(thi == k, xk, 0.0)
    dot = jnp.sum(dot_acc, axis=1, keepdims=True)

    colw = jax.lax.broadcasted_iota(jnp.int32, (rows, _LN), 1)
    lmask = colw == tlo
    xt = jnp.sum(jnp.where(lmask, xrow_acc, 0.0), axis=1, keepdims=True)
    gsh = (gp >> thi) & 1
    gt = jnp.sum(jnp.where(lmask, gsh, 0), axis=1, keepdims=True)
    gt = gt.astype(jnp.float32)
    cnt = jnp.sum(jax.lax.population_count(gp), axis=1,
                  keepdims=True).astype(jnp.float32)

    a = 1.0 - _EPS + _EPS / _K
    b = 2.0 * _EPS / _K
    c = _EPS / _K
    rowloss = -(a - b * gt) * (xt - lse) - c * (dot - cnt * lse)
    block_sum = jnp.sum(rowloss)

    @pl.when(pl.program_id(0) == 0)
    def _():
        out_ref[...] = jnp.zeros_like(out_ref)

    out_ref[...] = out_ref[...] + block_sum


def kernel(inputs, targets, nearest_map):
    bsz, n_cls = inputs.shape
    rows = 256 if bsz % 256 == 0 else bsz
    nblk = bsz // rows

    t2 = targets.reshape(nblk, rows, 1)
    packed = _pack(nearest_map)
    gathered = _make_sc_gather(bsz)(packed, targets)

    total = pl.pallas_call(
        functools.partial(_block_body, rows=rows, n_cls=n_cls),
        grid=(nblk,),
        in_specs=[
            pl.BlockSpec((rows, n_cls), lambda i: (i, 0)),
            pl.BlockSpec((1, rows, 1), lambda i: (i, 0, 0)),
            pl.BlockSpec((rows, _LN), lambda i: (i, 0)),
        ],
        out_specs=pl.BlockSpec((1, 1), lambda i: (0, 0)),
        out_shape=jax.ShapeDtypeStruct((1, 1), jnp.float32),
        compiler_params=pltpu.CompilerParams(
            dimension_semantics=("arbitrary",),
            vmem_limit_bytes=100 * 1024 * 1024,
        ),
    )(inputs, t2, gathered)

    return total[0, 0] * (1.0 / bsz)


# speedup vs baseline: 2.5916x; 1.3547x over previous
import functools

import jax
import jax.numpy as jnp
from jax import lax
from jax.experimental import pallas as pl
from jax.experimental.pallas import tpu as pltpu
from jax.experimental.pallas import tpu_sc as plsc

_EPS = 0.1
_K = 10
_LN = 128


def _pack_body(nm_ref, out_ref, *, n_words):
    acc = nm_ref[:, 0:_LN]
    for k in range(1, 32):
        acc = acc | (nm_ref[:, k * _LN:(k + 1) * _LN] << k)
    out_ref[...] = acc


def _pack(nearest_map):
    n_cls = nearest_map.shape[1]
    rv = 512
    return pl.pallas_call(
        functools.partial(_pack_body, n_words=_LN),
        grid=(nearest_map.shape[0] // rv,),
        in_specs=[pl.BlockSpec((rv, n_cls), lambda i: (i, 0))],
        out_specs=pl.BlockSpec((rv, _LN), lambda i: (i, 0)),
        out_shape=jax.ShapeDtypeStruct((nearest_map.shape[0], _LN), jnp.int32),
        compiler_params=pltpu.CompilerParams(
            dimension_semantics=("parallel",),
        ),
    )(nearest_map)


def _make_sc_gather(n_rows):
    info = plsc.get_sparse_core_info()
    nw = info.num_cores * info.num_subcores
    b_per_w = n_rows // nw
    chunk = 128
    n_chunks = b_per_w // chunk
    mesh = plsc.VectorSubcoreMesh(core_axis_name="c", subcore_axis_name="s")

    @functools.partial(
        pl.kernel, mesh=mesh,
        out_type=jax.ShapeDtypeStruct((n_rows, _LN), jnp.int32),
        scratch_types=[
            pltpu.VMEM((chunk,), jnp.int32),
            pltpu.VMEM((chunk, _LN), jnp.int32),
            pltpu.SemaphoreType.DMA,
        ],
    )
    def sc_gather(packed_hbm, t_hbm, out_hbm, idx_v, rows_v, sem):
        wid = lax.axis_index("s") * info.num_cores + lax.axis_index("c")
        base = wid * b_per_w

        def body(ci, carry):
            off = base + ci * chunk
            pltpu.sync_copy(t_hbm.at[pl.ds(off, chunk)], idx_v)
            pltpu.async_copy(packed_hbm.at[idx_v], rows_v, sem).wait()
            pltpu.sync_copy(rows_v, out_hbm.at[pl.ds(off, chunk)])
            return carry

        lax.fori_loop(0, n_chunks, body, 0)

    return sc_gather


def _block_body(x_ref, t2_ref, gp_ref, out_ref, *, rows, n_cls):
    x = x_ref[...]
    gp = gp_ref[...]
    tv = t2_ref[0]

    m = jnp.max(x, axis=1, keepdims=True)
    lse = m + jnp.log(jnp.sum(jnp.exp(x - m), axis=1, keepdims=True))

    col = jax.lax.broadcasted_iota(jnp.int32, (rows, n_cls), 1)
    mask = col == tv
    xt = jnp.sum(jnp.where(mask, x, 0.0), axis=1, keepdims=True)

    thi = tv >> 7
    tlo = tv & (_LN - 1)
    colw = jax.lax.broadcasted_iota(jnp.int32, (rows, _LN), 1)
    gsh = (gp >> thi) & 1
    gt = jnp.sum(jnp.where(colw == tlo, gsh, 0), axis=1, keepdims=True)
    gt = gt.astype(jnp.float32)

    dot_acc = jnp.zeros((rows, _LN), jnp.float32)
    for k in range(32):
        sel = (gp << (31 - k)) < 0
        dot_acc = dot_acc + jnp.where(sel, x[:, k * _LN:(k + 1) * _LN], 0.0)
    dot = jnp.sum(dot_acc, axis=1, keepdims=True)
    cnt = jnp.sum(jax.lax.population_count(gp), axis=1,
                  keepdims=True).astype(jnp.float32)

    a = 1.0 - _EPS + _EPS / _K
    b = 2.0 * _EPS / _K
    c = _EPS / _K
    rowloss = -(a - b * gt) * (xt - lse) - c * (dot - cnt * lse)
    block_sum = jnp.sum(rowloss)

    @pl.when(pl.program_id(0) == 0)
    def _():
        out_ref[...] = jnp.zeros_like(out_ref)

    out_ref[...] = out_ref[...] + block_sum


def kernel(inputs, targets, nearest_map):
    bsz, n_cls = inputs.shape
    rows = 512 if bsz % 512 == 0 else bsz
    nblk = bsz // rows

    t2 = targets.reshape(nblk, rows, 1)
    packed = _pack(nearest_map)
    gathered = _make_sc_gather(bsz)(packed, targets)

    total = pl.pallas_call(
        functools.partial(_block_body, rows=rows, n_cls=n_cls),
        grid=(nblk,),
        in_specs=[
            pl.BlockSpec((rows, n_cls), lambda i: (i, 0)),
            pl.BlockSpec((1, rows, 1), lambda i: (i, 0, 0)),
            pl.BlockSpec((rows, _LN), lambda i: (i, 0)),
        ],
        out_specs=pl.BlockSpec((1, 1), lambda i: (0, 0)),
        out_shape=jax.ShapeDtypeStruct((1, 1), jnp.float32),
        compiler_params=pltpu.CompilerParams(
            dimension_semantics=("arbitrary",),
            vmem_limit_bytes=100 * 1024 * 1024,
        ),
    )(inputs, t2, gathered)

    return total[0, 0] * (1.0 / bsz)


# speedup vs baseline: 2.6340x; 1.0164x over previous
import functools

import jax
import jax.numpy as jnp
from jax import lax
from jax.experimental import pallas as pl
from jax.experimental.pallas import tpu as pltpu
from jax.experimental.pallas import tpu_sc as plsc

_EPS = 0.1
_K = 10
_LN = 128


def _pack_body(nm_ref, out_ref, *, n_words):
    acc = nm_ref[:, 0:_LN]
    for k in range(1, 32):
        acc = acc | (nm_ref[:, k * _LN:(k + 1) * _LN] << k)
    out_ref[...] = acc


def _pack(nearest_map):
    n_cls = nearest_map.shape[1]
    rv = 512
    return pl.pallas_call(
        functools.partial(_pack_body, n_words=_LN),
        grid=(nearest_map.shape[0] // rv,),
        in_specs=[pl.BlockSpec((rv, n_cls), lambda i: (i, 0))],
        out_specs=pl.BlockSpec((rv, _LN), lambda i: (i, 0)),
        out_shape=jax.ShapeDtypeStruct((nearest_map.shape[0], _LN), jnp.int32),
        compiler_params=pltpu.CompilerParams(
            dimension_semantics=("parallel",),
        ),
    )(nearest_map)


def _make_sc_gather(n_rows):
    info = plsc.get_sparse_core_info()
    nw = info.num_cores * info.num_subcores
    b_per_w = n_rows // nw
    chunk = 128
    n_chunks = b_per_w // chunk
    mesh = plsc.VectorSubcoreMesh(core_axis_name="c", subcore_axis_name="s")

    @functools.partial(
        pl.kernel, mesh=mesh,
        out_type=jax.ShapeDtypeStruct((n_rows, _LN), jnp.int32),
        scratch_types=[
            pltpu.VMEM((chunk,), jnp.int32),
            pltpu.VMEM((chunk, _LN), jnp.int32),
            pltpu.SemaphoreType.DMA,
        ],
    )
    def sc_gather(packed_hbm, t_hbm, out_hbm, idx_v, rows_v, sem):
        wid = lax.axis_index("s") * info.num_cores + lax.axis_index("c")
        base = wid * b_per_w

        def body(ci, carry):
            off = base + ci * chunk
            pltpu.sync_copy(t_hbm.at[pl.ds(off, chunk)], idx_v)
            pltpu.async_copy(packed_hbm.at[idx_v], rows_v, sem).wait()
            pltpu.sync_copy(rows_v, out_hbm.at[pl.ds(off, chunk)])
            return carry

        lax.fori_loop(0, n_chunks, body, 0)

    return sc_gather


def _block_body(x_ref, t2_ref, gp_ref, out_ref, *, rows, n_cls):
    x = x_ref[...]
    gp = gp_ref[...]
    tv = t2_ref[0]

    m = jnp.max(x, axis=1, keepdims=True)
    lse = m + jnp.log(jnp.sum(jnp.exp(x - m), axis=1, keepdims=True))

    col = jax.lax.broadcasted_iota(jnp.int32, (rows, n_cls), 1)
    mask = col == tv
    xt = jnp.sum(jnp.where(mask, x, 0.0), axis=1, keepdims=True)

    thi = tv >> 7
    tlo = tv & (_LN - 1)
    colw = jax.lax.broadcasted_iota(jnp.int32, (rows, _LN), 1)
    gsh = (gp >> thi) & 1
    gt = jnp.sum(jnp.where(colw == tlo, gsh, 0), axis=1, keepdims=True)
    gt = gt.astype(jnp.float32)

    dot_acc = jnp.zeros((rows, _LN), jnp.float32)
    for k in range(32):
        sel = (gp << (31 - k)) < 0
        dot_acc = dot_acc + jnp.where(sel, x[:, k * _LN:(k + 1) * _LN], 0.0)
    dot = jnp.sum(dot_acc, axis=1, keepdims=True)
    cnt = jnp.sum(jax.lax.population_count(gp), axis=1,
                  keepdims=True).astype(jnp.float32)

    a = 1.0 - _EPS + _EPS / _K
    b = 2.0 * _EPS / _K
    c = _EPS / _K
    rowloss = -(a - b * gt) * (xt - lse) - c * (dot - cnt * lse)
    block_sum = jnp.sum(rowloss)

    @pl.when(pl.program_id(0) == 0)
    def _():
        out_ref[...] = jnp.zeros_like(out_ref)

    out_ref[...] = out_ref[...] + block_sum


def kernel(inputs, targets, nearest_map):
    bsz, n_cls = inputs.shape
    rows = 1024 if bsz % 1024 == 0 else bsz
    nblk = bsz // rows

    t2 = targets.reshape(nblk, rows, 1)
    packed = _pack(nearest_map)
    gathered = _make_sc_gather(bsz)(packed, targets)

    total = pl.pallas_call(
        functools.partial(_block_body, rows=rows, n_cls=n_cls),
        grid=(nblk,),
        in_specs=[
            pl.BlockSpec((rows, n_cls), lambda i: (i, 0)),
            pl.BlockSpec((1, rows, 1), lambda i: (i, 0, 0)),
            pl.BlockSpec((rows, _LN), lambda i: (i, 0)),
        ],
        out_specs=pl.BlockSpec((1, 1), lambda i: (0, 0)),
        out_shape=jax.ShapeDtypeStruct((1, 1), jnp.float32),
        compiler_params=pltpu.CompilerParams(
            dimension_semantics=("arbitrary",),
            vmem_limit_bytes=100 * 1024 * 1024,
        ),
    )(inputs, t2, gathered)

    return total[0, 0] * (1.0 / bsz)


# speedup vs baseline: 2.7227x; 1.0337x over previous
import functools

import jax
import jax.numpy as jnp
from jax import lax
from jax.experimental import pallas as pl
from jax.experimental.pallas import tpu as pltpu
from jax.experimental.pallas import tpu_sc as plsc

_EPS = 0.1
_K = 10
_LN = 128


def _pack_body(nm_ref, out_ref, *, n_words):
    acc = nm_ref[:, 0:_LN]
    for k in range(1, 32):
        acc = acc | (nm_ref[:, k * _LN:(k + 1) * _LN] << k)
    out_ref[...] = acc


def _pack(nearest_map):
    n_cls = nearest_map.shape[1]
    rv = 512
    return pl.pallas_call(
        functools.partial(_pack_body, n_words=_LN),
        grid=(nearest_map.shape[0] // rv,),
        in_specs=[pl.BlockSpec((rv, n_cls), lambda i: (i, 0))],
        out_specs=pl.BlockSpec((rv, _LN), lambda i: (i, 0)),
        out_shape=jax.ShapeDtypeStruct((nearest_map.shape[0], _LN), jnp.int32),
        compiler_params=pltpu.CompilerParams(
            dimension_semantics=("parallel",),
        ),
    )(nearest_map)


def _make_sc_gather(n_rows):
    info = plsc.get_sparse_core_info()
    nw = info.num_cores * info.num_subcores
    b_per_w = n_rows // nw
    chunk = 128
    n_chunks = b_per_w // chunk
    mesh = plsc.VectorSubcoreMesh(core_axis_name="c", subcore_axis_name="s")

    @functools.partial(
        pl.kernel, mesh=mesh,
        out_type=jax.ShapeDtypeStruct((n_rows, _LN), jnp.int32),
        scratch_types=[
            pltpu.VMEM((chunk,), jnp.int32),
            pltpu.VMEM((chunk, _LN), jnp.int32),
            pltpu.SemaphoreType.DMA,
        ],
    )
    def sc_gather(packed_hbm, t_hbm, out_hbm, idx_v, rows_v, sem):
        wid = lax.axis_index("s") * info.num_cores + lax.axis_index("c")
        base = wid * b_per_w

        def body(ci, carry):
            off = base + ci * chunk
            pltpu.sync_copy(t_hbm.at[pl.ds(off, chunk)], idx_v)
            pltpu.async_copy(packed_hbm.at[idx_v], rows_v, sem).wait()
            pltpu.sync_copy(rows_v, out_hbm.at[pl.ds(off, chunk)])
            return carry

        lax.fori_loop(0, n_chunks, body, 0)

    return sc_gather


def _block_body(x_ref, t2_ref, gp_ref, out_ref, *, rows, n_cls):
    x = x_ref[...]
    gp = gp_ref[...]
    tv = t2_ref[0]

    m = jnp.max(x, axis=1, keepdims=True)

    col = jax.lax.broadcasted_iota(jnp.int32, (rows, n_cls), 1)
    mask = col == tv
    xt = jnp.sum(jnp.where(mask, x, 0.0), axis=1, keepdims=True)

    thi = tv >> 7
    tlo = tv & (_LN - 1)
    colw = jax.lax.broadcasted_iota(jnp.int32, (rows, _LN), 1)
    gsh = (gp >> thi) & 1
    gt = jnp.sum(jnp.where(colw == tlo, gsh, 0), axis=1, keepdims=True)
    gt = gt.astype(jnp.float32)

    s_acc = jnp.zeros((rows, _LN), jnp.float32)
    dot_acc = jnp.zeros((rows, _LN), jnp.float32)
    for k in range(32):
        xk = x[:, k * _LN:(k + 1) * _LN]
        s_acc = s_acc + jnp.exp(xk - m)
        sel = (gp << (31 - k)) < 0
        dot_acc = dot_acc + jnp.where(sel, xk, 0.0)
    lse = m + jnp.log(jnp.sum(s_acc, axis=1, keepdims=True))
    dot = jnp.sum(dot_acc, axis=1, keepdims=True)
    cnt = jnp.sum(jax.lax.population_count(gp), axis=1,
                  keepdims=True).astype(jnp.float32)

    a = 1.0 - _EPS + _EPS / _K
    b = 2.0 * _EPS / _K
    c = _EPS / _K
    rowloss = -(a - b * gt) * (xt - lse) - c * (dot - cnt * lse)
    block_sum = jnp.sum(rowloss)

    @pl.when(pl.program_id(0) == 0)
    def _():
        out_ref[...] = jnp.zeros_like(out_ref)

    out_ref[...] = out_ref[...] + block_sum


def kernel(inputs, targets, nearest_map):
    bsz, n_cls = inputs.shape
    rows = 1024 if bsz % 1024 == 0 else bsz
    nblk = bsz // rows

    t2 = targets.reshape(nblk, rows, 1)
    packed = _pack(nearest_map)
    gathered = _make_sc_gather(bsz)(packed, targets)

    total = pl.pallas_call(
        functools.partial(_block_body, rows=rows, n_cls=n_cls),
        grid=(nblk,),
        in_specs=[
            pl.BlockSpec((rows, n_cls), lambda i: (i, 0)),
            pl.BlockSpec((1, rows, 1), lambda i: (i, 0, 0)),
            pl.BlockSpec((rows, _LN), lambda i: (i, 0)),
        ],
        out_specs=pl.BlockSpec((1, 1), lambda i: (0, 0)),
        out_shape=jax.ShapeDtypeStruct((1, 1), jnp.float32),
        compiler_params=pltpu.CompilerParams(
            dimension_semantics=("arbitrary",),
            vmem_limit_bytes=100 * 1024 * 1024,
        ),
    )(inputs, t2, gathered)

    return total[0, 0] * (1.0 / bsz)
